# Initial kernel scaffold; baseline (speedup 1.0000x reference)
import functools

import jax
import jax.numpy as jnp
from jax import lax
from jax.experimental import pallas as pl
from jax.experimental.pallas import tpu as pltpu
from jax.experimental.pallas import tpu_sc as plsc

N = 10000
E = 320000
NFEAT = 128
NHID = 256
NCLASS = 64
NLAYERS = 8

NSUB = 16
HALF = NCLASS // 2
EPT = E // NSUB
CHUNK = 128
NFULL = EPT // CHUNK
TAIL = EPT - NFULL * CHUNK
ROWS_PT = N // NSUB
BM = 1000


def _dense_in_body(x_ref, w_in_ref, b_in_ref, w_out_ref, out_ref):
    h = jnp.dot(x_ref[...], w_in_ref[...], preferred_element_type=jnp.float32)
    h = jnp.maximum(h + b_in_ref[...], 0.0)
    out_ref[...] = jnp.dot(h, w_out_ref[...], preferred_element_type=jnp.float32)


def _dense_in(x, w_in, b_in, w_out):
    return pl.pallas_call(
        _dense_in_body,
        grid=(N // BM,),
        in_specs=[
            pl.BlockSpec((BM, NFEAT), lambda i: (i, 0)),
            pl.BlockSpec((NFEAT, NHID), lambda i: (0, 0)),
            pl.BlockSpec((1, NHID), lambda i: (0, 0)),
            pl.BlockSpec((NHID, NCLASS), lambda i: (0, 0)),
        ],
        out_specs=pl.BlockSpec((BM, NCLASS), lambda i: (i, 0)),
        out_shape=jax.ShapeDtypeStruct((N, NCLASS), jnp.float32),
    )(x, w_in, b_in, w_out)


def _softmax_body(g_ref, b_ref, out_ref):
    y = g_ref[...] + b_ref[...]
    m = jnp.max(y, axis=1, keepdims=True)
    z = y - m
    lse = jnp.log(jnp.sum(jnp.exp(z), axis=1, keepdims=True))
    out_ref[...] = z - lse


def _softmax(g, b_out):
    return pl.pallas_call(
        _softmax_body,
        grid=(N // BM,),
        in_specs=[
            pl.BlockSpec((BM, NCLASS), lambda i: (i, 0)),
            pl.BlockSpec((1, NCLASS), lambda i: (0, 0)),
        ],
        out_specs=pl.BlockSpec((BM, NCLASS), lambda i: (i, 0)),
        out_shape=jax.ShapeDtypeStruct((N, NCLASS), jnp.float32),
    )(g, b_out)


def _sweep_chunk(gin_hbm, acc_sh, src2_hbm, dst_hbm, w_hbm,
                 src_v, dst_v, w_v, rows_v, sem, e_src, e_loc, n):
    pltpu.sync_copy(src2_hbm.at[pl.ds(e_src, n)], src_v)
    pltpu.sync_copy(dst_hbm.at[pl.ds(e_loc, n)], dst_v)
    pltpu.sync_copy(w_hbm.at[pl.ds(e_loc, n)], w_v)
    pltpu.async_copy(gin_hbm.at[src_v], rows_v, sem).wait()

    def edge_body(e, carry):
        wb = plsc.load_gather(w_v, [jnp.full((16,), 0, jnp.int32) + e])
        r0 = rows_v[e, pl.ds(0, 16)] * wb
        r1 = rows_v[e, pl.ds(16, 16)] * wb
        rows_v[e, pl.ds(0, 16)] = r0
        rows_v[e, pl.ds(16, 16)] = r1
        return carry

    lax.fori_loop(0, n, edge_body, 0)
    pltpu.sync_copy(rows_v, acc_sh.at[dst_v], add=True)


def _prop(g0, src2, dst, w, a1p, a2p):
    mesh = plsc.VectorSubcoreMesh(core_axis_name="c", subcore_axis_name="s")

    @functools.partial(
        pl.kernel,
        mesh=mesh,
        out_type=[
            jax.ShapeDtypeStruct((2 * N, HALF), jnp.float32),
            jax.ShapeDtypeStruct((2 * N, HALF), jnp.float32),
        ],
        scratch_types=[
            pltpu.VMEM_SHARED((N, HALF), jnp.float32),
            pltpu.VMEM((ROWS_PT, HALF), jnp.float32),
            pltpu.VMEM((ROWS_PT, HALF), jnp.float32),
            pltpu.VMEM((ROWS_PT, HALF), jnp.float32),
            pltpu.VMEM((CHUNK,), jnp.int32),
            pltpu.VMEM((CHUNK,), jnp.int32),
            pltpu.VMEM((CHUNK,), jnp.float32),
            pltpu.VMEM((CHUNK, HALF), jnp.float32),
            pltpu.VMEM((TAIL,), jnp.int32),
            pltpu.VMEM((TAIL,), jnp.int32),
            pltpu.VMEM((TAIL,), jnp.float32),
            pltpu.VMEM((TAIL, HALF), jnp.float32),
            pltpu.VMEM((16,), jnp.float32),
            pltpu.VMEM((16,), jnp.float32),
            pltpu.SemaphoreType.DMA,
        ],
    )
    def prop_kernel(g0_hbm, src2_hbm, dst_hbm, w_hbm, a1_hbm, a2_hbm,
                    out_q, out_p, acc_sh, g0_v, comb_v, zero_v,
                    src_v, dst_v, w_v, rows_v,
                    src_t, dst_t, w_t, rows_t, a1_v, a2_v, sem):
        c = lax.axis_index("c")
        s = lax.axis_index("s")
        row0 = s * ROWS_PT
        edge0 = s * EPT
        gbase = c * N + row0

        pltpu.sync_copy(a1_hbm, a1_v)
        pltpu.sync_copy(a2_hbm, a2_v)
        pltpu.sync_copy(g0_hbm.at[pl.ds(gbase, ROWS_PT)], g0_v)

        def zero_body(i, carry):
            zero_v[i, pl.ds(0, 16)] = jnp.zeros((16,), jnp.float32)
            zero_v[i, pl.ds(16, 16)] = jnp.zeros((16,), jnp.float32)
            return carry

        lax.fori_loop(0, ROWS_PT, zero_body, 0)
        pltpu.sync_copy(zero_v, acc_sh.at[pl.ds(row0, ROWS_PT)])
        plsc.subcore_barrier()

        def do_layer(l, gin_hbm, gout_hbm):
            def chunk_body(k, carry):
                _sweep_chunk(gin_hbm, acc_sh, src2_hbm, dst_hbm, w_hbm,
                             src_v, dst_v, w_v, rows_v, sem,
                             c * E + edge0 + k * CHUNK,
                             edge0 + k * CHUNK, CHUNK)
                return carry

            lax.fori_loop(0, NFULL, chunk_body, 0)
            _sweep_chunk(gin_hbm, acc_sh, src2_hbm, dst_hbm, w_hbm,
                         src_t, dst_t, w_t, rows_t, sem,
                         c * E + edge0 + NFULL * CHUNK,
                         edge0 + NFULL * CHUNK, TAIL)
            plsc.subcore_barrier()

            pltpu.sync_copy(acc_sh.at[pl.ds(row0, ROWS_PT)], comb_v)
            a1b = plsc.load_gather(a1_v, [jnp.full((16,), l, jnp.int32)])
            a2b = plsc.load_gather(a2_v, [jnp.full((16,), l, jnp.int32)])

            def comb_body(i, carry):
                for j in (0, 16):
                    v = comb_v[i, pl.ds(j, 16)] * a1b + g0_v[i, pl.ds(j, 16)] * a2b
                    comb_v[i, pl.ds(j, 16)] = v
                return carry

            lax.fori_loop(0, ROWS_PT, comb_body, 0)
            pltpu.sync_copy(comb_v, gout_hbm.at[pl.ds(gbase, ROWS_PT)])
            pltpu.sync_copy(zero_v, acc_sh.at[pl.ds(row0, ROWS_PT)])
            plsc.subcore_barrier()

        bufs = [g0_hbm] + [out_p if (l % 2 == 0) else out_q for l in range(NLAYERS)]
        for l in range(NLAYERS):
            do_layer(l, bufs[l], bufs[l + 1])

    return prop_kernel(g0, src2, dst, w, a1p, a2p)


def kernel(x, edge_index, edge_weight, W_in, b_in, W_out, b_out, alpha1, alpha2):
    g0 = _dense_in(x, W_in, b_in.reshape(1, NHID), W_out)
    g0_split = g0.reshape(N, 2, HALF).transpose(1, 0, 2).reshape(2 * N, HALF)

    src = edge_index[1].astype(jnp.int32)
    dst = edge_index[0].astype(jnp.int32)
    src2 = jnp.concatenate([src, src + N])
    a1p = jnp.pad(alpha1, (0, 16 - NLAYERS))
    a2p = jnp.pad(alpha2, (0, 16 - NLAYERS))

    q, _ = _prop(g0_split, src2, dst, edge_weight, a1p, a2p)
    g = q.reshape(2, N, HALF).transpose(1, 0, 2).reshape(N, NCLASS)
    return _softmax(g, b_out.reshape(1, NCLASS))


# speedup vs baseline: 4.1441x; 4.1441x over previous
import functools

import jax
import jax.numpy as jnp
from jax import lax
from jax.experimental import pallas as pl
from jax.experimental.pallas import tpu as pltpu
from jax.experimental.pallas import tpu_sc as plsc

N = 10000
E = 320000
NFEAT = 128
NHID = 256
NCLASS = 64
NLAYERS = 8

NSUB = 16
HALF = NCLASS // 2
EPT = E // NSUB
CHUNK = 128
NFULL = EPT // CHUNK
TAIL = EPT - NFULL * CHUNK
NP = 10240
ROWS_PT = NP // NSUB
BM = 1000


def _dense_in_body(x_ref, w_in_ref, b_in_ref, w_out_ref, out_ref):
    h = jnp.dot(x_ref[...], w_in_ref[...], preferred_element_type=jnp.float32)
    h = jnp.maximum(h + b_in_ref[...], 0.0)
    out_ref[...] = jnp.dot(h, w_out_ref[...], preferred_element_type=jnp.float32)


def _dense_in(x, w_in, b_in, w_out):
    return pl.pallas_call(
        _dense_in_body,
        grid=(N // BM,),
        in_specs=[
            pl.BlockSpec((BM, NFEAT), lambda i: (i, 0)),
            pl.BlockSpec((NFEAT, NHID), lambda i: (0, 0)),
            pl.BlockSpec((1, NHID), lambda i: (0, 0)),
            pl.BlockSpec((NHID, NCLASS), lambda i: (0, 0)),
        ],
        out_specs=pl.BlockSpec((BM, NCLASS), lambda i: (i, 0)),
        out_shape=jax.ShapeDtypeStruct((N, NCLASS), jnp.float32),
    )(x, w_in, b_in, w_out)


def _softmax_body(g_ref, b_ref, out_ref):
    y = g_ref[...] + b_ref[...]
    m = jnp.max(y, axis=1, keepdims=True)
    z = y - m
    lse = jnp.log(jnp.sum(jnp.exp(z), axis=1, keepdims=True))
    out_ref[...] = z - lse


def _softmax(g, b_out):
    return pl.pallas_call(
        _softmax_body,
        grid=(N // BM,),
        in_specs=[
            pl.BlockSpec((BM, NCLASS), lambda i: (i, 0)),
            pl.BlockSpec((1, NCLASS), lambda i: (0, 0)),
        ],
        out_specs=pl.BlockSpec((BM, NCLASS), lambda i: (i, 0)),
        out_shape=jax.ShapeDtypeStruct((N, NCLASS), jnp.float32),
    )(g, b_out)


def _sweep_chunk(gin_hbm, acc_sh, src2_hbm, dst_hbm, w_hbm,
                 src_v, dst_v, w_v, rows_v, sem, e_src, e_loc, n):
    pltpu.sync_copy(src2_hbm.at[pl.ds(e_src, n)], src_v)
    pltpu.sync_copy(dst_hbm.at[pl.ds(e_loc, n)], dst_v)
    pltpu.sync_copy(w_hbm.at[pl.ds(e_loc, n)], w_v)
    pltpu.async_copy(gin_hbm.at[src_v], rows_v, sem).wait()

    def edge_body(e, carry):
        wb = plsc.load_gather(w_v, [jnp.full((16,), 0, jnp.int32) + e])
        r0 = rows_v[e, pl.ds(0, 16)] * wb
        r1 = rows_v[e, pl.ds(16, 16)] * wb
        rows_v[e, pl.ds(0, 16)] = r0
        rows_v[e, pl.ds(16, 16)] = r1
        return carry

    lax.fori_loop(0, n, edge_body, 0)
    pltpu.sync_copy(rows_v, acc_sh.at[dst_v], add=True)


def _prop(g0, src2, dst, w, a1p, a2p):
    mesh = plsc.VectorSubcoreMesh(core_axis_name="c", subcore_axis_name="s")

    @functools.partial(
        pl.kernel,
        mesh=mesh,
        compiler_params=pltpu.CompilerParams(
            needs_layout_passes=False, use_tc_tiling_on_sc=False),
        out_type=[
            jax.ShapeDtypeStruct((2 * NP, HALF), jnp.float32),
            jax.ShapeDtypeStruct((2 * NP, HALF), jnp.float32),
        ],
        scratch_types=[
            pltpu.VMEM_SHARED((NP, HALF), jnp.float32),
            pltpu.VMEM((ROWS_PT, HALF), jnp.float32),
            pltpu.VMEM((ROWS_PT, HALF), jnp.float32),
            pltpu.VMEM((ROWS_PT, HALF), jnp.float32),
            pltpu.VMEM((CHUNK,), jnp.int32),
            pltpu.VMEM((CHUNK,), jnp.int32),
            pltpu.VMEM((CHUNK,), jnp.float32),
            pltpu.VMEM((CHUNK, HALF), jnp.float32),
            pltpu.VMEM((TAIL,), jnp.int32),
            pltpu.VMEM((TAIL,), jnp.int32),
            pltpu.VMEM((TAIL,), jnp.float32),
            pltpu.VMEM((TAIL, HALF), jnp.float32),
            pltpu.VMEM((16, 16), jnp.float32),
            pltpu.VMEM((16, 16), jnp.float32),
            pltpu.SemaphoreType.DMA,
        ],
    )
    def prop_kernel(g0_hbm, src2_hbm, dst_hbm, w_hbm, a1_hbm, a2_hbm,
                    out_q, out_p, acc_sh, g0_v, comb_v, zero_v,
                    src_v, dst_v, w_v, rows_v,
                    src_t, dst_t, w_t, rows_t, a1_v, a2_v, sem):
        c = lax.axis_index("c")
        s = lax.axis_index("s")
        row0 = s * ROWS_PT
        edge0 = s * EPT
        gbase = c * NP + row0

        pltpu.sync_copy(a1_hbm, a1_v)
        pltpu.sync_copy(a2_hbm, a2_v)
        pltpu.sync_copy(g0_hbm.at[pl.ds(gbase, ROWS_PT)], g0_v)

        def zero_body(i, carry):
            zero_v[i, pl.ds(0, 16)] = jnp.zeros((16,), jnp.float32)
            zero_v[i, pl.ds(16, 16)] = jnp.zeros((16,), jnp.float32)
            return carry

        lax.fori_loop(0, ROWS_PT, zero_body, 0)
        pltpu.sync_copy(zero_v, acc_sh.at[pl.ds(row0, ROWS_PT)])
        plsc.subcore_barrier()

        def do_layer(l, gin_hbm, gout_hbm):
            def chunk_body(k, carry):
                _sweep_chunk(gin_hbm, acc_sh, src2_hbm, dst_hbm, w_hbm,
                             src_v, dst_v, w_v, rows_v, sem,
                             c * E + edge0 + k * CHUNK,
                             edge0 + k * CHUNK, CHUNK)
                return carry

            lax.fori_loop(0, NFULL, chunk_body, 0)
            _sweep_chunk(gin_hbm, acc_sh, src2_hbm, dst_hbm, w_hbm,
                         src_t, dst_t, w_t, rows_t, sem,
                         c * E + edge0 + NFULL * CHUNK,
                         edge0 + NFULL * CHUNK, TAIL)
            plsc.subcore_barrier()

            pltpu.sync_copy(acc_sh.at[pl.ds(row0, ROWS_PT)], comb_v)
            a1b = a1_v[l, pl.ds(0, 16)]
            a2b = a2_v[l, pl.ds(0, 16)]

            def comb_body(i, carry):
                for j in (0, 16):
                    v = comb_v[i, pl.ds(j, 16)] * a1b + g0_v[i, pl.ds(j, 16)] * a2b
                    comb_v[i, pl.ds(j, 16)] = v
                return carry

            lax.fori_loop(0, ROWS_PT, comb_body, 0)
            pltpu.sync_copy(comb_v, gout_hbm.at[pl.ds(gbase, ROWS_PT)])
            pltpu.sync_copy(zero_v, acc_sh.at[pl.ds(row0, ROWS_PT)])
            plsc.subcore_barrier()

        bufs = [g0_hbm] + [out_p if (l % 2 == 0) else out_q for l in range(NLAYERS)]
        for l in range(NLAYERS):
            do_layer(l, bufs[l], bufs[l + 1])

    return prop_kernel(g0, src2, dst, w, a1p, a2p)


def kernel(x, edge_index, edge_weight, W_in, b_in, W_out, b_out, alpha1, alpha2):
    g0 = _dense_in(x, W_in, b_in.reshape(1, NHID), W_out)
    g0_pad = jnp.pad(g0, ((0, NP - N), (0, 0)))
    g0_split = g0_pad.reshape(NP, 2, HALF).transpose(1, 0, 2).reshape(2 * NP, HALF)

    src = edge_index[1].astype(jnp.int32)
    dst = edge_index[0].astype(jnp.int32)
    src2 = jnp.concatenate([src, src + NP])
    a1p = jnp.tile(jnp.pad(alpha1, (0, 16 - NLAYERS)).reshape(16, 1), (1, 16))
    a2p = jnp.tile(jnp.pad(alpha2, (0, 16 - NLAYERS)).reshape(16, 1), (1, 16))

    q, _ = _prop(g0_split, src2, dst, edge_weight, a1p, a2p)
    g = q.reshape(2, NP, HALF)[:, :N].transpose(1, 0, 2).reshape(N, NCLASS)
    return _softmax(g, b_out.reshape(1, NCLASS))


# speedup vs baseline: 7.4932x; 1.8082x over previous
import functools

import jax
import jax.numpy as jnp
from jax import lax
from jax.experimental import pallas as pl
from jax.experimental.pallas import tpu as pltpu
from jax.experimental.pallas import tpu_sc as plsc

N = 10000
E = 320000
NFEAT = 128
NHID = 256
NCLASS = 64
NLAYERS = 8

NSUB = 16
HALF = NCLASS // 2
CW = 128
SUP = 4
E2 = 327680
RPT = E2 // NSUB // CW
NSUP = RPT // SUP
NSUP2 = NSUP // 2
NP = 10240
ROWS_PT = NP // NSUB
BM = 1000


def _dense_in_body(x_ref, w_in_ref, b_in_ref, w_out_ref, out_ref):
    h = jnp.dot(x_ref[...], w_in_ref[...], preferred_element_type=jnp.float32)
    h = jnp.maximum(h + b_in_ref[...], 0.0)
    out_ref[...] = jnp.dot(h, w_out_ref[...], preferred_element_type=jnp.float32)


def _dense_in(x, w_in, b_in, w_out):
    return pl.pallas_call(
        _dense_in_body,
        grid=(N // BM,),
        in_specs=[
            pl.BlockSpec((BM, NFEAT), lambda i: (i, 0)),
            pl.BlockSpec((NFEAT, NHID), lambda i: (0, 0)),
            pl.BlockSpec((1, NHID), lambda i: (0, 0)),
            pl.BlockSpec((NHID, NCLASS), lambda i: (0, 0)),
        ],
        out_specs=pl.BlockSpec((BM, NCLASS), lambda i: (i, 0)),
        out_shape=jax.ShapeDtypeStruct((N, NCLASS), jnp.float32),
    )(x, w_in, b_in, w_out)


def _softmax_body(g_ref, b_ref, out_ref):
    y = g_ref[...] + b_ref[...]
    m = jnp.max(y, axis=1, keepdims=True)
    z = y - m
    lse = jnp.log(jnp.sum(jnp.exp(z), axis=1, keepdims=True))
    out_ref[...] = z - lse


def _softmax(g, b_out):
    return pl.pallas_call(
        _softmax_body,
        grid=(N // BM,),
        in_specs=[
            pl.BlockSpec((BM, NCLASS), lambda i: (i, 0)),
            pl.BlockSpec((1, NCLASS), lambda i: (0, 0)),
        ],
        out_specs=pl.BlockSpec((BM, NCLASS), lambda i: (i, 0)),
        out_shape=jax.ShapeDtypeStruct((N, NCLASS), jnp.float32),
    )(g, b_out)


def _prop(g0, src2, dst2, w, a1p, a2p):
    mesh = plsc.VectorSubcoreMesh(core_axis_name="c", subcore_axis_name="s")

    @functools.partial(
        pl.kernel,
        mesh=mesh,
        compiler_params=pltpu.CompilerParams(
            needs_layout_passes=False, use_tc_tiling_on_sc=False),
        out_type=[
            jax.ShapeDtypeStruct((2 * NP, HALF), jnp.float32),
            jax.ShapeDtypeStruct((2 * NP, HALF), jnp.float32),
        ],
        scratch_types=[
            pltpu.VMEM_SHARED((NP, HALF), jnp.float32),
            pltpu.VMEM((ROWS_PT, HALF), jnp.float32),
            pltpu.VMEM((ROWS_PT, HALF), jnp.float32),
            pltpu.VMEM((ROWS_PT, HALF), jnp.float32),
            pltpu.VMEM((SUP, CW), jnp.int32),
            pltpu.VMEM((SUP, CW), jnp.int32),
            pltpu.VMEM((SUP, CW), jnp.int32),
            pltpu.VMEM((SUP, CW), jnp.int32),
            pltpu.VMEM((SUP * CW,), jnp.float32),
            pltpu.VMEM((SUP * CW,), jnp.float32),
            pltpu.VMEM((SUP * CW, HALF), jnp.float32),
            pltpu.VMEM((SUP * CW, HALF), jnp.float32),
            pltpu.VMEM((16, 16), jnp.float32),
            pltpu.VMEM((16, 16), jnp.float32),
            pltpu.SemaphoreType.DMA,
            pltpu.SemaphoreType.DMA,
            pltpu.SemaphoreType.DMA,
            pltpu.SemaphoreType.DMA,
        ],
    )
    def prop_kernel(g0_hbm, src2_hbm, dst2_hbm, w_hbm, a1_hbm, a2_hbm,
                    out_q, out_p, acc_sh, g0_v, comb_v, zero_v,
                    srcA, srcB, dstA, dstB, wA, wB, rowsA, rowsB,
                    a1_v, a2_v, gsA, gsB, ssA, ssB):
        c = lax.axis_index("c")
        s = lax.axis_index("s")
        row0 = s * ROWS_PT
        gbase = c * NP + row0
        rb_loc = s * RPT
        rb_src = c * (E2 // CW) + rb_loc

        def load_idx(cc, srcx, dstx, wx):
            pltpu.sync_copy(src2_hbm.at[pl.ds(rb_src + cc * SUP, SUP)], srcx)
            pltpu.sync_copy(dst2_hbm.at[pl.ds(rb_loc + cc * SUP, SUP)], dstx)
            pltpu.sync_copy(w_hbm.at[pl.ds((rb_loc + cc * SUP) * CW, SUP * CW)], wx)

        def gather(gin, srcx, rowsx, sem):
            for j in range(SUP):
                pltpu.async_copy(gin.at[srcx.at[j]],
                                 rowsx.at[pl.ds(j * CW, CW)], sem)

        def wait_gather(gin, srcx, rowsx, sem):
            for j in range(SUP):
                pltpu.make_async_copy(gin.at[srcx.at[j]],
                                      rowsx.at[pl.ds(j * CW, CW)], sem).wait()

        def scatter(rowsx, dstx, sem):
            for j in range(SUP):
                pltpu.async_copy(rowsx.at[pl.ds(j * CW, CW)],
                                 acc_sh.at[dstx.at[j]], sem, add=True)

        def wait_scatter(rowsx, dstx, sem):
            for j in range(SUP):
                pltpu.make_async_copy(rowsx.at[pl.ds(j * CW, CW)],
                                      acc_sh.at[dstx.at[j]], sem).wait()

        def multiply(rowsx, wx):
            def body(k, carry):
                for u in range(4):
                    e = k * 4 + u
                    wb = plsc.load_gather(wx, [jnp.full((16,), 0, jnp.int32) + e])
                    rowsx[e, pl.ds(0, 16)] = rowsx[e, pl.ds(0, 16)] * wb
                    rowsx[e, pl.ds(16, 16)] = rowsx[e, pl.ds(16, 16)] * wb
                return carry

            lax.fori_loop(0, SUP * CW // 4, body, 0)

        pltpu.sync_copy(a1_hbm, a1_v)
        pltpu.sync_copy(a2_hbm, a2_v)
        pltpu.sync_copy(g0_hbm.at[pl.ds(gbase, ROWS_PT)], g0_v)

        def zero_body(i, carry):
            zero_v[i, pl.ds(0, 16)] = jnp.zeros((16,), jnp.float32)
            zero_v[i, pl.ds(16, 16)] = jnp.zeros((16,), jnp.float32)
            return carry

        lax.fori_loop(0, ROWS_PT, zero_body, 0)
        pltpu.sync_copy(zero_v, acc_sh.at[pl.ds(row0, ROWS_PT)])
        plsc.subcore_barrier()

        def do_layer(l, gin, gout):
            load_idx(0, srcA, dstA, wA)
            gather(gin, srcA, rowsA, gsA)

            def iter_body(k2, carry):
                wait_gather(gin, srcA, rowsA, gsA)

                @pl.when(k2 > 0)
                def _():
                    wait_scatter(rowsB, dstB, ssB)

                load_idx(2 * k2 + 1, srcB, dstB, wB)
                gather(gin, srcB, rowsB, gsB)
                multiply(rowsA, wA)
                scatter(rowsA, dstA, ssA)

                wait_gather(gin, srcB, rowsB, gsB)
                wait_scatter(rowsA, dstA, ssA)

                @pl.when(k2 < NSUP2 - 1)
                def _():
                    load_idx(2 * k2 + 2, srcA, dstA, wA)
                    gather(gin, srcA, rowsA, gsA)

                multiply(rowsB, wB)
                scatter(rowsB, dstB, ssB)
                return carry

            lax.fori_loop(0, NSUP2, iter_body, 0)
            wait_scatter(rowsB, dstB, ssB)
            plsc.subcore_barrier()

            pltpu.sync_copy(acc_sh.at[pl.ds(row0, ROWS_PT)], comb_v)
            a1b = a1_v[l, pl.ds(0, 16)]
            a2b = a2_v[l, pl.ds(0, 16)]

            def comb_body(i, carry):
                for j in (0, 16):
                    v = comb_v[i, pl.ds(j, 16)] * a1b + g0_v[i, pl.ds(j, 16)] * a2b
                    comb_v[i, pl.ds(j, 16)] = v
                return carry

            lax.fori_loop(0, ROWS_PT, comb_body, 0)
            pltpu.sync_copy(comb_v, gout.at[pl.ds(gbase, ROWS_PT)])
            pltpu.sync_copy(zero_v, acc_sh.at[pl.ds(row0, ROWS_PT)])
            plsc.subcore_barrier()

        bufs = [g0_hbm] + [out_p if (l % 2 == 0) else out_q for l in range(NLAYERS)]
        for l in range(NLAYERS):
            do_layer(l, bufs[l], bufs[l + 1])

    return prop_kernel(g0, src2, dst2, w, a1p, a2p)


def kernel(x, edge_index, edge_weight, W_in, b_in, W_out, b_out, alpha1, alpha2):
    g0 = _dense_in(x, W_in, b_in.reshape(1, NHID), W_out)
    g0_pad = jnp.pad(g0, ((0, NP - N), (0, 0)))
    g0_split = g0_pad.reshape(NP, 2, HALF).transpose(1, 0, 2).reshape(2 * NP, HALF)

    src = edge_index[1].astype(jnp.int32)
    dst = edge_index[0].astype(jnp.int32)
    src_p = jnp.pad(src, (0, E2 - E))
    dst_p = jnp.pad(dst, (0, E2 - E), constant_values=N)
    w_p = jnp.pad(edge_weight, (0, E2 - E))
    src2 = jnp.concatenate([src_p, src_p + NP]).reshape(2 * E2 // CW, CW)
    dst2 = dst_p.reshape(E2 // CW, CW)
    a1p = jnp.tile(jnp.pad(alpha1, (0, 16 - NLAYERS)).reshape(16, 1), (1, 16))
    a2p = jnp.tile(jnp.pad(alpha2, (0, 16 - NLAYERS)).reshape(16, 1), (1, 16))

    q, _ = _prop(g0_split, src2, dst2, w_p, a1p, a2p)
    g = q.reshape(2, NP, HALF)[:, :N].transpose(1, 0, 2).reshape(N, NCLASS)
    return _softmax(g, b_out.reshape(1, NCLASS))


# speedup vs baseline: 9.0957x; 1.2139x over previous
import functools

import jax
import jax.numpy as jnp
from jax import lax
from jax.experimental import pallas as pl
from jax.experimental.pallas import tpu as pltpu
from jax.experimental.pallas import tpu_sc as plsc

N = 10000
E = 320000
NFEAT = 128
NHID = 256
NCLASS = 64
NLAYERS = 8

NSUB = 16
HALF = NCLASS // 2
CW = 128
SUP = 4
E2 = 327680
RPT = E2 // NSUB // CW
NSUP = RPT // SUP
NSUP2 = NSUP // 2
NP = 10240
ROWS_PT = NP // NSUB
ZR = ROWS_PT // 4
BM = 1000


def _dense_in_body(x_ref, w_in_ref, b_in_ref, w_out_ref, out_ref):
    h = jnp.dot(x_ref[...], w_in_ref[...], preferred_element_type=jnp.float32)
    h = jnp.maximum(h + b_in_ref[...], 0.0)
    out_ref[...] = jnp.dot(h, w_out_ref[...], preferred_element_type=jnp.float32)


def _dense_in(x, w_in, b_in, w_out):
    return pl.pallas_call(
        _dense_in_body,
        grid=(N // BM,),
        in_specs=[
            pl.BlockSpec((BM, NFEAT), lambda i: (i, 0)),
            pl.BlockSpec((NFEAT, NHID), lambda i: (0, 0)),
            pl.BlockSpec((1, NHID), lambda i: (0, 0)),
            pl.BlockSpec((NHID, NCLASS), lambda i: (0, 0)),
        ],
        out_specs=pl.BlockSpec((BM, NCLASS), lambda i: (i, 0)),
        out_shape=jax.ShapeDtypeStruct((N, NCLASS), jnp.float32),
    )(x, w_in, b_in, w_out)


def _softmax_body(g_ref, b_ref, out_ref):
    y = g_ref[...] + b_ref[...]
    m = jnp.max(y, axis=1, keepdims=True)
    z = y - m
    lse = jnp.log(jnp.sum(jnp.exp(z), axis=1, keepdims=True))
    out_ref[...] = z - lse


def _softmax(g, b_out):
    return pl.pallas_call(
        _softmax_body,
        grid=(N // BM,),
        in_specs=[
            pl.BlockSpec((BM, NCLASS), lambda i: (i, 0)),
            pl.BlockSpec((1, NCLASS), lambda i: (0, 0)),
        ],
        out_specs=pl.BlockSpec((BM, NCLASS), lambda i: (i, 0)),
        out_shape=jax.ShapeDtypeStruct((N, NCLASS), jnp.float32),
    )(g, b_out)


def _prop(g0, src1, dst2, w, a1p, a2p):
    mesh = plsc.VectorSubcoreMesh(core_axis_name="c", subcore_axis_name="s")

    @functools.partial(
        pl.kernel,
        mesh=mesh,
        compiler_params=pltpu.CompilerParams(
            needs_layout_passes=False, use_tc_tiling_on_sc=False),
        out_type=[
            jax.ShapeDtypeStruct((2 * NP, HALF), jnp.float32),
        ],
        scratch_types=[
            pltpu.VMEM_SHARED((NP, HALF), jnp.float32),
            pltpu.VMEM_SHARED((NP, HALF), jnp.float32),
            pltpu.VMEM((ROWS_PT, HALF), jnp.float32),
            pltpu.VMEM((ROWS_PT, HALF), jnp.float32),
            pltpu.VMEM((ZR, HALF), jnp.float32),
            pltpu.VMEM((SUP, CW), jnp.int32),
            pltpu.VMEM((SUP, CW), jnp.int32),
            pltpu.VMEM((SUP, CW), jnp.int32),
            pltpu.VMEM((SUP, CW), jnp.int32),
            pltpu.VMEM((SUP * CW,), jnp.float32),
            pltpu.VMEM((SUP * CW,), jnp.float32),
            pltpu.VMEM((SUP * CW, HALF), jnp.float32),
            pltpu.VMEM((SUP * CW, HALF), jnp.float32),
            pltpu.VMEM((16, 16), jnp.float32),
            pltpu.VMEM((16, 16), jnp.float32),
            pltpu.SemaphoreType.DMA,
            pltpu.SemaphoreType.DMA,
            pltpu.SemaphoreType.DMA,
            pltpu.SemaphoreType.DMA,
        ],
    )
    def prop_kernel(g0_hbm, src1_hbm, dst2_hbm, w_hbm, a1_hbm, a2_hbm,
                    out_q, gA_sh, gB_sh, g0_v, comb_v, zero_v,
                    srcA, srcB, dstA, dstB, wA, wB, rowsA, rowsB,
                    a1_v, a2_v, gsA, gsB, ssA, ssB):
        c = lax.axis_index("c")
        s = lax.axis_index("s")
        row0 = s * ROWS_PT
        gbase = c * NP + row0
        rb_loc = s * RPT

        def load_idx(cc, srcx, dstx, wx):
            pltpu.sync_copy(src1_hbm.at[pl.ds(rb_loc + cc * SUP, SUP)], srcx)
            pltpu.sync_copy(dst2_hbm.at[pl.ds(rb_loc + cc * SUP, SUP)], dstx)
            pltpu.sync_copy(w_hbm.at[pl.ds((rb_loc + cc * SUP) * CW, SUP * CW)], wx)

        def gather(gin, srcx, rowsx, sem):
            for j in range(SUP):
                pltpu.async_copy(gin.at[srcx.at[j]],
                                 rowsx.at[pl.ds(j * CW, CW)], sem)

        def wait_gather(gin, srcx, rowsx, sem):
            for j in range(SUP):
                pltpu.make_async_copy(gin.at[srcx.at[j]],
                                      rowsx.at[pl.ds(j * CW, CW)], sem).wait()

        def scatter(gacc, rowsx, dstx, sem):
            for j in range(SUP):
                pltpu.async_copy(rowsx.at[pl.ds(j * CW, CW)],
                                 gacc.at[dstx.at[j]], sem, add=True)

        def wait_scatter(gacc, rowsx, dstx, sem):
            for j in range(SUP):
                pltpu.make_async_copy(rowsx.at[pl.ds(j * CW, CW)],
                                      gacc.at[dstx.at[j]], sem).wait()

        def multiply(rowsx, wx):
            def body(k, carry):
                for u in range(4):
                    e = k * 4 + u
                    wb = plsc.load_gather(wx, [jnp.full((16,), 0, jnp.int32) + e])
                    rowsx[e, pl.ds(0, 16)] = rowsx[e, pl.ds(0, 16)] * wb
                    rowsx[e, pl.ds(16, 16)] = rowsx[e, pl.ds(16, 16)] * wb
                return carry

            lax.fori_loop(0, SUP * CW // 4, body, 0)

        pltpu.sync_copy(a1_hbm, a1_v)
        pltpu.sync_copy(a2_hbm, a2_v)
        pltpu.sync_copy(g0_hbm.at[pl.ds(gbase, ROWS_PT)], g0_v)
        pltpu.sync_copy(g0_v, gA_sh.at[pl.ds(row0, ROWS_PT)])

        def zero_body(i, carry):
            zero_v[i, pl.ds(0, 16)] = jnp.zeros((16,), jnp.float32)
            zero_v[i, pl.ds(16, 16)] = jnp.zeros((16,), jnp.float32)
            return carry

        lax.fori_loop(0, ZR, zero_body, 0)

        def zero_slice(dst_sh):
            for z in range(ROWS_PT // ZR):
                pltpu.sync_copy(zero_v, dst_sh.at[pl.ds(row0 + z * ZR, ZR)])

        zero_slice(gB_sh)
        plsc.subcore_barrier()

        def do_layer(l, gin, gacc, last):
            load_idx(0, srcA, dstA, wA)
            gather(gin, srcA, rowsA, gsA)

            def iter_body(k2, carry):
                wait_gather(gin, srcA, rowsA, gsA)

                @pl.when(k2 > 0)
                def _():
                    wait_scatter(gacc, rowsB, dstB, ssB)

                load_idx(2 * k2 + 1, srcB, dstB, wB)
                gather(gin, srcB, rowsB, gsB)
                multiply(rowsA, wA)
                scatter(gacc, rowsA, dstA, ssA)

                wait_gather(gin, srcB, rowsB, gsB)
                wait_scatter(gacc, rowsA, dstA, ssA)

                @pl.when(k2 < NSUP2 - 1)
                def _():
                    load_idx(2 * k2 + 2, srcA, dstA, wA)
                    gather(gin, srcA, rowsA, gsA)

                multiply(rowsB, wB)
                scatter(gacc, rowsB, dstB, ssB)
                return carry

            lax.fori_loop(0, NSUP2, iter_body, 0)
            wait_scatter(gacc, rowsB, dstB, ssB)
            plsc.subcore_barrier()

            pltpu.sync_copy(gacc.at[pl.ds(row0, ROWS_PT)], comb_v)
            a1b = a1_v[l, pl.ds(0, 16)]
            a2b = a2_v[l, pl.ds(0, 16)]

            def comb_body(i, carry):
                for j in (0, 16):
                    v = comb_v[i, pl.ds(j, 16)] * a1b + g0_v[i, pl.ds(j, 16)] * a2b
                    comb_v[i, pl.ds(j, 16)] = v
                return carry

            lax.fori_loop(0, ROWS_PT, comb_body, 0)
            if last:
                pltpu.sync_copy(comb_v, out_q.at[pl.ds(gbase, ROWS_PT)])
            else:
                pltpu.sync_copy(comb_v, gacc.at[pl.ds(row0, ROWS_PT)])
                zero_slice(gin)
                plsc.subcore_barrier()

        for l in range(NLAYERS):
            gin = gA_sh if (l % 2 == 0) else gB_sh
            gacc = gB_sh if (l % 2 == 0) else gA_sh
            do_layer(l, gin, gacc, l == NLAYERS - 1)

    return prop_kernel(g0, src1, dst2, w, a1p, a2p)


def kernel(x, edge_index, edge_weight, W_in, b_in, W_out, b_out, alpha1, alpha2):
    g0 = _dense_in(x, W_in, b_in.reshape(1, NHID), W_out)
    g0_pad = jnp.pad(g0, ((0, NP - N), (0, 0)))
    g0_split = g0_pad.reshape(NP, 2, HALF).transpose(1, 0, 2).reshape(2 * NP, HALF)

    src = edge_index[1].astype(jnp.int32)
    dst = edge_index[0].astype(jnp.int32)
    src_p = jnp.pad(src, (0, E2 - E))
    dst_p = jnp.pad(dst, (0, E2 - E), constant_values=N)
    w_p = jnp.pad(edge_weight, (0, E2 - E))
    src1 = src_p.reshape(E2 // CW, CW)
    dst2 = dst_p.reshape(E2 // CW, CW)
    a1p = jnp.tile(jnp.pad(alpha1, (0, 16 - NLAYERS)).reshape(16, 1), (1, 16))
    a2p = jnp.tile(jnp.pad(alpha2, (0, 16 - NLAYERS)).reshape(16, 1), (1, 16))

    q, = _prop(g0_split, src1, dst2, w_p, a1p, a2p)
    g = q.reshape(2, NP, HALF)[:, :N].transpose(1, 0, 2).reshape(N, NCLASS)
    return _softmax(g, b_out.reshape(1, NCLASS))


# speedup vs baseline: 10.7591x; 1.1829x over previous
import functools

import jax
import jax.numpy as jnp
from jax import lax
from jax.experimental import pallas as pl
from jax.experimental.pallas import tpu as pltpu
from jax.experimental.pallas import tpu_sc as plsc

N = 10000
E = 320000
NFEAT = 128
NHID = 256
NCLASS = 64
NLAYERS = 8

NSUB = 16
HALF = NCLASS // 2
CW = 128
SUP = 4
E2 = 327680
RPT = E2 // NSUB // CW
NSUP = RPT // SUP
NSUP2 = NSUP // 2
NP = 10240
ROWS_PT = NP // NSUB
ZR = ROWS_PT // 4
BM = 1000


def _dense_in_body(x_ref, w_in_ref, b_in_ref, w_out_ref, out_ref):
    h = jnp.dot(x_ref[...], w_in_ref[...], preferred_element_type=jnp.float32)
    h = jnp.maximum(h + b_in_ref[...], 0.0)
    out_ref[...] = jnp.dot(h, w_out_ref[...], preferred_element_type=jnp.float32)


def _dense_in(x, w_in, b_in, w_out):
    return pl.pallas_call(
        _dense_in_body,
        grid=(N // BM,),
        in_specs=[
            pl.BlockSpec((BM, NFEAT), lambda i: (i, 0)),
            pl.BlockSpec((NFEAT, NHID), lambda i: (0, 0)),
            pl.BlockSpec((1, NHID), lambda i: (0, 0)),
            pl.BlockSpec((NHID, NCLASS), lambda i: (0, 0)),
        ],
        out_specs=pl.BlockSpec((BM, NCLASS), lambda i: (i, 0)),
        out_shape=jax.ShapeDtypeStruct((N, NCLASS), jnp.float32),
    )(x, w_in, b_in, w_out)


def _softmax_body(g_ref, b_ref, out_ref):
    y = g_ref[...] + b_ref[...]
    m = jnp.max(y, axis=1, keepdims=True)
    z = y - m
    lse = jnp.log(jnp.sum(jnp.exp(z), axis=1, keepdims=True))
    out_ref[...] = z - lse


def _softmax(g, b_out):
    return pl.pallas_call(
        _softmax_body,
        grid=(N // BM,),
        in_specs=[
            pl.BlockSpec((BM, NCLASS), lambda i: (i, 0)),
            pl.BlockSpec((1, NCLASS), lambda i: (0, 0)),
        ],
        out_specs=pl.BlockSpec((BM, NCLASS), lambda i: (i, 0)),
        out_shape=jax.ShapeDtypeStruct((N, NCLASS), jnp.float32),
    )(g, b_out)


def _prop(g0, src1, dst2, w, a1p, a2p):
    mesh = plsc.VectorSubcoreMesh(core_axis_name="c", subcore_axis_name="s")

    @functools.partial(
        pl.kernel,
        mesh=mesh,
        compiler_params=pltpu.CompilerParams(
            needs_layout_passes=False, use_tc_tiling_on_sc=False),
        out_type=[
            jax.ShapeDtypeStruct((2 * NP, HALF), jnp.float32),
        ],
        scratch_types=[
            pltpu.VMEM_SHARED((NP, HALF), jnp.float32),
            pltpu.VMEM_SHARED((NP, HALF), jnp.float32),
            pltpu.VMEM((ROWS_PT, HALF), jnp.float32),
            pltpu.VMEM((ZR, HALF), jnp.float32),
            pltpu.VMEM((SUP, CW), jnp.int32),
            pltpu.VMEM((SUP, CW), jnp.int32),
            pltpu.VMEM((SUP, CW), jnp.int32),
            pltpu.VMEM((SUP, CW), jnp.int32),
            pltpu.VMEM((SUP, CW), jnp.int32),
            pltpu.VMEM((SUP, CW), jnp.int32),
            pltpu.VMEM((SUP * CW,), jnp.float32),
            pltpu.VMEM((SUP * CW,), jnp.float32),
            pltpu.VMEM((SUP * CW,), jnp.float32),
            pltpu.VMEM((SUP * CW, HALF), jnp.float32),
            pltpu.VMEM((SUP * CW, HALF), jnp.float32),
            pltpu.VMEM((SUP * CW, HALF), jnp.float32),
            pltpu.VMEM((16, 16), jnp.float32),
            pltpu.VMEM((16, 16), jnp.float32),
            pltpu.SemaphoreType.DMA,
            pltpu.SemaphoreType.DMA,
            pltpu.SemaphoreType.DMA,
            pltpu.SemaphoreType.DMA,
            pltpu.SemaphoreType.DMA,
            pltpu.SemaphoreType.DMA,
        ],
    )
    def prop_kernel(g0_hbm, src1_hbm, dst2_hbm, w_hbm, a1_hbm, a2_hbm,
                    out_q, gA_sh, gB_sh, g0_v, zero_v,
                    src0, src1v, src2v, dst0, dst1, dst2v, w0, w1, w2,
                    rows0, rows1, rows2,
                    a1_v, a2_v, gs0, gs1, gs2, ss0, ss1, ss2):
        c = lax.axis_index("c")
        s = lax.axis_index("s")
        row0 = s * ROWS_PT
        gbase = c * NP + row0
        rb_loc = s * RPT

        SRC = (src0, src1v, src2v)
        DST = (dst0, dst1, dst2v)
        WGT = (w0, w1, w2)
        ROWS = (rows0, rows1, rows2)
        GS = (gs0, gs1, gs2)
        SS = (ss0, ss1, ss2)

        def load_idx(cc, r):
            pltpu.sync_copy(src1_hbm.at[pl.ds(rb_loc + cc * SUP, SUP)], SRC[r])
            pltpu.sync_copy(dst2_hbm.at[pl.ds(rb_loc + cc * SUP, SUP)], DST[r])
            pltpu.sync_copy(w_hbm.at[pl.ds((rb_loc + cc * SUP) * CW, SUP * CW)], WGT[r])

        def gather(gin, r):
            for j in range(SUP):
                pltpu.async_copy(gin.at[SRC[r].at[j]],
                                 ROWS[r].at[pl.ds(j * CW, CW)], GS[r])

        def wait_gather(gin, r):
            for j in range(SUP):
                pltpu.make_async_copy(gin.at[SRC[r].at[j]],
                                      ROWS[r].at[pl.ds(j * CW, CW)], GS[r]).wait()

        def scatter(gacc, r):
            for j in range(SUP):
                pltpu.async_copy(ROWS[r].at[pl.ds(j * CW, CW)],
                                 gacc.at[DST[r].at[j]], SS[r], add=True)

        def wait_scatter(gacc, r):
            for j in range(SUP):
                pltpu.make_async_copy(ROWS[r].at[pl.ds(j * CW, CW)],
                                      gacc.at[DST[r].at[j]], SS[r]).wait()

        def multiply(r):
            rowsx, wx = ROWS[r], WGT[r]

            def body(k, carry):
                for u in range(4):
                    e = k * 4 + u
                    wb = plsc.load_gather(wx, [jnp.full((16,), 0, jnp.int32) + e])
                    rowsx[e, pl.ds(0, 16)] = rowsx[e, pl.ds(0, 16)] * wb
                    rowsx[e, pl.ds(16, 16)] = rowsx[e, pl.ds(16, 16)] * wb
                return carry

            lax.fori_loop(0, SUP * CW // 4, body, 0)

        pltpu.sync_copy(a1_hbm, a1_v)
        pltpu.sync_copy(a2_hbm, a2_v)
        pltpu.sync_copy(g0_hbm.at[pl.ds(gbase, ROWS_PT)], g0_v)
        pltpu.sync_copy(g0_v, gA_sh.at[pl.ds(row0, ROWS_PT)])

        def zero_body(i, carry):
            zero_v[i, pl.ds(0, 16)] = jnp.zeros((16,), jnp.float32)
            zero_v[i, pl.ds(16, 16)] = jnp.zeros((16,), jnp.float32)
            return carry

        lax.fori_loop(0, ZR, zero_body, 0)

        def zero_slice(dst_sh):
            for z in range(ROWS_PT // ZR):
                pltpu.sync_copy(zero_v, dst_sh.at[pl.ds(row0 + z * ZR, ZR)])

        zero_slice(gB_sh)
        plsc.subcore_barrier()

        def do_layer(l, gin, gacc, last):
            load_idx(0, 0)
            gather(gin, 0)
            load_idx(1, 1)
            gather(gin, 1)

            def process(ct, r, rn):
                @pl.when(ct < NSUP)
                def _():
                    wait_gather(gin, r)
                    multiply(r)

                    @pl.when(ct >= 1)
                    def _():
                        wait_scatter(gacc, rn)

                    @pl.when(ct + 2 < NSUP)
                    def _():
                        load_idx(ct + 2, rn)
                        gather(gin, rn)

                    scatter(gacc, r)

            def iter_body(k3, carry):
                for off in range(3):
                    process(k3 * 3 + off, off, (off + 2) % 3)
                return carry

            lax.fori_loop(0, (NSUP + 3) // 3 + 1, iter_body, 0)
            wait_scatter(gacc, (NSUP - 1) % 3)
            plsc.subcore_barrier()

            a1b = a1_v[l, pl.ds(0, 16)]
            a2b = a2_v[l, pl.ds(0, 16)]

            def comb_pass(buf, base, nrows):
                pltpu.sync_copy(gacc.at[pl.ds(row0 + base, nrows)],
                                buf.at[pl.ds(0, nrows)])

                def comb_body(i, carry):
                    for j in (0, 16):
                        v = buf[i, pl.ds(j, 16)] * a1b \
                            + g0_v[base + i, pl.ds(j, 16)] * a2b
                        buf[i, pl.ds(j, 16)] = v
                    return carry

                lax.fori_loop(0, nrows, comb_body, 0)
                if last:
                    pltpu.sync_copy(buf.at[pl.ds(0, nrows)],
                                    out_q.at[pl.ds(gbase + base, nrows)])
                else:
                    pltpu.sync_copy(buf.at[pl.ds(0, nrows)],
                                    gacc.at[pl.ds(row0 + base, nrows)])

            comb_pass(rows0, 0, SUP * CW)
            comb_pass(rows1, SUP * CW, ROWS_PT - SUP * CW)
            if not last:
                zero_slice(gin)
                plsc.subcore_barrier()

        for l in range(NLAYERS):
            gin = gA_sh if (l % 2 == 0) else gB_sh
            gacc = gB_sh if (l % 2 == 0) else gA_sh
            do_layer(l, gin, gacc, l == NLAYERS - 1)

    return prop_kernel(g0, src1, dst2, w, a1p, a2p)


def kernel(x, edge_index, edge_weight, W_in, b_in, W_out, b_out, alpha1, alpha2):
    g0 = _dense_in(x, W_in, b_in.reshape(1, NHID), W_out)
    g0_pad = jnp.pad(g0, ((0, NP - N), (0, 0)))
    g0_split = g0_pad.reshape(NP, 2, HALF).transpose(1, 0, 2).reshape(2 * NP, HALF)

    src = edge_index[1].astype(jnp.int32)
    dst = edge_index[0].astype(jnp.int32)
    src_p = jnp.pad(src, (0, E2 - E))
    dst_p = jnp.pad(dst, (0, E2 - E), constant_values=N)
    w_p = jnp.pad(edge_weight, (0, E2 - E))
    src1 = src_p.reshape(E2 // CW, CW)
    dst2 = dst_p.reshape(E2 // CW, CW)
    a1p = jnp.tile(jnp.pad(alpha1, (0, 16 - NLAYERS)).reshape(16, 1), (1, 16))
    a2p = jnp.tile(jnp.pad(alpha2, (0, 16 - NLAYERS)).reshape(16, 1), (1, 16))

    q, = _prop(g0_split, src1, dst2, w_p, a1p, a2p)
    g = q.reshape(2, NP, HALF)[:, :N].transpose(1, 0, 2).reshape(N, NCLASS)
    return _softmax(g, b_out.reshape(1, NCLASS))
